# Initial kernel scaffold; baseline (speedup 1.0000x reference)
import jax
import jax.numpy as jnp
from jax.experimental import pallas as pl


def kernel(node_features, edge_index, edge_features, batch, node_W, node_b, node_g, node_beta, edge_W, edge_b, edge_g, edge_beta, msg_W1, msg_b1, msg_g, msg_beta, msg_W2, msg_b2, gru_Wi, gru_Wh, gru_bi, gru_bh, ln_g, ln_beta, out_W, out_b, out_g, out_beta):
    raise NotImplementedError("write your pallas kernel here")


# speedup vs baseline: 2.8761x; 2.8761x over previous
import functools

import jax
import jax.numpy as jnp
from jax import lax
from jax.experimental import pallas as pl
from jax.experimental.pallas import tpu as pltpu
from jax.experimental.pallas import tpu_sc as plsc

_NW = 32
_CH = 128


def _ln(h, g, b):
    mu = jnp.mean(h, axis=-1, keepdims=True)
    var = jnp.mean((h - mu) ** 2, axis=-1, keepdims=True)
    return (h - mu) * lax.rsqrt(var + 1e-5) * g + b


def _enc_body(xf_ref, w_ref, b_ref, g_ref, beta_ref, o_ref):
    h = jnp.dot(xf_ref[...], w_ref[...], preferred_element_type=jnp.float32)
    o_ref[...] = jnp.maximum(_ln(h + b_ref[...], g_ref[...], beta_ref[...]), 0.0)


def _encode(xf, w, b, g, beta, blk):
    n, din = xf.shape
    h = w.shape[1]
    return pl.pallas_call(
        _enc_body,
        grid=(n // blk,),
        in_specs=[
            pl.BlockSpec((blk, din), lambda i: (i, 0)),
            pl.BlockSpec((din, h), lambda i: (0, 0)),
            pl.BlockSpec((1, h), lambda i: (0, 0)),
            pl.BlockSpec((1, h), lambda i: (0, 0)),
            pl.BlockSpec((1, h), lambda i: (0, 0)),
        ],
        out_specs=pl.BlockSpec((blk, h), lambda i: (i, 0)),
        out_shape=jax.ShapeDtypeStruct((n, h), jnp.float32),
    )(xf, w, b.reshape(1, -1), g.reshape(1, -1), beta.reshape(1, -1))


def _mlp_body(xi_ref, xj_ref, ea_ref, w1a, w1b, w1c, b1, g, beta, w2, b2, o_ref):
    h = (jnp.dot(xi_ref[...], w1a[...], preferred_element_type=jnp.float32)
         + jnp.dot(xj_ref[...], w1b[...], preferred_element_type=jnp.float32)
         + jnp.dot(ea_ref[...], w1c[...], preferred_element_type=jnp.float32))
    h = jnp.maximum(_ln(h + b1[...], g[...], beta[...]), 0.0)
    o_ref[...] = jnp.dot(h, w2[...], preferred_element_type=jnp.float32) + b2[...]


def _edge_mlp(xi, xj, ea, w1, b1, g, beta, w2, b2, blk):
    e, h = xi.shape
    vspec = pl.BlockSpec((1, h), lambda i: (0, 0))
    wspec = pl.BlockSpec((h, h), lambda i: (0, 0))
    espec = pl.BlockSpec((blk, h), lambda i: (i, 0))
    return pl.pallas_call(
        _mlp_body,
        grid=(e // blk,),
        in_specs=[espec, espec, espec, wspec, wspec, wspec, vspec, vspec,
                  vspec, wspec, vspec],
        out_specs=espec,
        out_shape=jax.ShapeDtypeStruct((e, h), jnp.float32),
    )(xi, xj, ea, w1[:h], w1[h:2 * h], w1[2 * h:], b1.reshape(1, -1),
      g.reshape(1, -1), beta.reshape(1, -1), w2, b2.reshape(1, -1))


def _gru_body(a0, a1, x_ref, wi, wh, bi, bh, g, beta, o_ref):
    x = x_ref[...]
    h = x.shape[1]
    gi = jnp.dot(a0[...] + a1[...], wi[...],
                 preferred_element_type=jnp.float32) + bi[...]
    gh = jnp.dot(x, wh[...], preferred_element_type=jnp.float32) + bh[...]
    r = jax.nn.sigmoid(gi[:, :h] + gh[:, :h])
    z = jax.nn.sigmoid(gi[:, h:2 * h] + gh[:, h:2 * h])
    ng = jnp.tanh(gi[:, 2 * h:] + r * gh[:, 2 * h:])
    xn = (1.0 - z) * ng + z * x
    o_ref[...] = _ln(xn + x, g[...], beta[...])


def _gru(a0, a1, x, wi, wh, bi, bh, g, beta, blk):
    n, h = x.shape
    nspec = pl.BlockSpec((blk, h), lambda i: (i, 0))
    wspec = pl.BlockSpec((h, 3 * h), lambda i: (0, 0))
    b3spec = pl.BlockSpec((1, 3 * h), lambda i: (0, 0))
    vspec = pl.BlockSpec((1, h), lambda i: (0, 0))
    return pl.pallas_call(
        _gru_body,
        grid=(n // blk,),
        in_specs=[nspec, nspec, nspec, wspec, wspec, b3spec, b3spec,
                  vspec, vspec],
        out_specs=nspec,
        out_shape=jax.ShapeDtypeStruct((n, h), jnp.float32),
    )(a0, a1, x, wi, wh, bi.reshape(1, -1), bh.reshape(1, -1),
      g.reshape(1, -1), beta.reshape(1, -1))


def _seg_body(x_ref, b_ref, sum_ref, max_ref, cnt_ref, *, nseg, blk):
    @pl.when(pl.program_id(0) == 0)
    def _():
        sum_ref[...] = jnp.zeros_like(sum_ref)
        cnt_ref[...] = jnp.zeros_like(cnt_ref)
        max_ref[...] = jnp.full_like(max_ref, -jnp.inf)

    x = x_ref[...]
    bi = b_ref[...]
    gids = lax.broadcasted_iota(jnp.int32, (1, nseg), 1)
    oh = (bi == gids).astype(jnp.float32)
    sum_ref[...] += lax.dot_general(oh, x, (((0,), (0,)), ((), ())),
                                    preferred_element_type=jnp.float32)
    cnt_ref[...] += jnp.sum(oh, axis=0)[:, None]
    lo = bi[0, 0]
    hi = bi[blk - 1, 0]

    def mx(gg, carry):
        mg = jnp.max(jnp.where(bi == gg, x, -jnp.inf), axis=0, keepdims=True)
        max_ref[pl.ds(gg, 1), :] = jnp.maximum(max_ref[pl.ds(gg, 1), :], mg)
        return carry

    lax.fori_loop(lo, hi + 1, mx, 0)


def _segment_reduce(x, batch, nseg, blk):
    n, h = x.shape
    ospec = pl.BlockSpec((nseg, h), lambda i: (0, 0))
    return pl.pallas_call(
        functools.partial(_seg_body, nseg=nseg, blk=blk),
        grid=(n // blk,),
        in_specs=[
            pl.BlockSpec((blk, h), lambda i: (i, 0)),
            pl.BlockSpec((blk, 1), lambda i: (i, 0)),
        ],
        out_specs=[ospec, ospec, ospec],
        out_shape=[jax.ShapeDtypeStruct((nseg, h), jnp.float32)] * 3,
    )(x, batch.reshape(n, 1))


def _proj_body(sum_ref, max_ref, cnt_ref, wa, wb, b, g, beta, o_ref):
    mean = sum_ref[...] / jnp.maximum(cnt_ref[...], 1.0)
    h = (jnp.dot(mean, wa[...], preferred_element_type=jnp.float32)
         + jnp.dot(max_ref[...], wb[...], preferred_element_type=jnp.float32))
    o_ref[...] = jnp.maximum(_ln(h + b[...], g[...], beta[...]), 0.0)


def _project(s, mx, cnt, w, b, g, beta):
    nseg, h = s.shape
    h2 = w.shape[1]
    return pl.pallas_call(
        _proj_body,
        out_shape=jax.ShapeDtypeStruct((nseg, h2), jnp.float32),
    )(s, mx, cnt, w[:h], w[h:], b.reshape(1, -1), g.reshape(1, -1),
      beta.reshape(1, -1))


def _sc_gather(x, src, dst):
    n, h = x.shape
    e = src.shape[0]
    nchunk = e // _CH
    iters = (nchunk + _NW - 1) // _NW

    @functools.partial(
        pl.kernel,
        mesh=plsc.VectorSubcoreMesh(core_axis_name="c", subcore_axis_name="s"),
        out_type=[jax.ShapeDtypeStruct((e, h), jnp.float32),
                  jax.ShapeDtypeStruct((e, h), jnp.float32)],
        scratch_types=[pltpu.VMEM((_CH,), jnp.int32),
                       pltpu.VMEM((_CH,), jnp.int32),
                       pltpu.VMEM((_CH, h), jnp.float32),
                       pltpu.VMEM((_CH, h), jnp.float32),
                       pltpu.SemaphoreType.DMA,
                       pltpu.SemaphoreType.DMA],
    )
    def k(x_hbm, src_hbm, dst_hbm, xj_hbm, xi_hbm, sidx, didx, srows, drows,
          sem1, sem2):
        wid = lax.axis_index("s") * 2 + lax.axis_index("c")

        def body(i, carry):
            cid = i * _NW + wid

            @pl.when(cid < nchunk)
            def _():
                base = cid * _CH
                pltpu.sync_copy(src_hbm.at[pl.ds(base, _CH)], sidx)
                pltpu.sync_copy(dst_hbm.at[pl.ds(base, _CH)], didx)
                cp1 = pltpu.async_copy(x_hbm.at[sidx], srows, sem1)
                cp2 = pltpu.async_copy(x_hbm.at[didx], drows, sem2)
                cp1.wait()
                cp2.wait()
                pltpu.sync_copy(srows, xj_hbm.at[pl.ds(base, _CH)])
                pltpu.sync_copy(drows, xi_hbm.at[pl.ds(base, _CH)])

            return carry

        lax.fori_loop(0, iters, body, 0)

    return k(x, src, dst)


def _sc_scatter_add(m, dst, zeros):
    e, h = m.shape
    n = zeros.shape[0]
    nchunk = e // _CH
    iters = (nchunk + _NW - 1) // _NW
    zr = 80
    nzc = n // zr
    ziters = (nzc + 15) // 16

    @functools.partial(
        pl.kernel,
        mesh=plsc.VectorSubcoreMesh(core_axis_name="c", subcore_axis_name="s"),
        out_type=jax.ShapeDtypeStruct((2, n, h), jnp.float32),
        scratch_types=[pltpu.VMEM((1, _CH), jnp.int32),
                       pltpu.VMEM((_CH, h), jnp.float32),
                       pltpu.VMEM_SHARED((n, h), jnp.float32)],
    )
    def k(m_hbm, dst_hbm, z_hbm, out_hbm, didx, rows, acc):
        c = lax.axis_index("c")
        s = lax.axis_index("s")
        wid = s * 2 + c

        def zbody(i, carry):
            zc = i * 16 + s

            @pl.when(zc < nzc)
            def _():
                pltpu.sync_copy(z_hbm.at[pl.ds(zc * zr, zr)],
                                acc.at[pl.ds(zc * zr, zr)])

            return carry

        lax.fori_loop(0, ziters, zbody, 0)
        plsc.subcore_barrier()

        def body(i, carry):
            cid = i * _NW + wid

            @pl.when(cid < nchunk)
            def _():
                base = cid * _CH
                pltpu.sync_copy(dst_hbm.at[pl.ds(base, _CH)], didx.at[0])
                pltpu.sync_copy(m_hbm.at[pl.ds(base, _CH)], rows)
                pltpu.sync_copy(rows, acc.at[didx.at[0]], add=True)

            return carry

        lax.fori_loop(0, iters, body, 0)
        plsc.subcore_barrier()

        def obody(i, carry):
            zc = i * 16 + s

            @pl.when(zc < nzc)
            def _():
                pltpu.sync_copy(acc.at[pl.ds(zc * zr, zr)],
                                out_hbm.at[c, pl.ds(zc * zr, zr)])

            return carry

        lax.fori_loop(0, ziters, obody, 0)

    return k(m, dst, zeros)


def kernel(node_features, edge_index, edge_features, batch,
           node_W, node_b, node_g, node_beta,
           edge_W, edge_b, edge_g, edge_beta,
           msg_W1, msg_b1, msg_g, msg_beta, msg_W2, msg_b2,
           gru_Wi, gru_Wh, gru_bi, gru_bh, ln_g, ln_beta,
           out_W, out_b, out_g, out_beta):
    n = node_features.shape[0]
    h = node_W.shape[1]
    nlayers = msg_W1.shape[0]
    nseg = 64
    src = edge_index[0]
    dst = edge_index[1]

    x = _encode(node_features, node_W, node_b, node_g, node_beta, blk=2000)
    ea = _encode(edge_features, edge_W, edge_b, edge_g, edge_beta, blk=4000)
    zeros = jnp.zeros((n, h), jnp.float32)

    for l in range(nlayers):
        xj, xi = _sc_gather(x, src, dst)
        m = _edge_mlp(xi, xj, ea, msg_W1[l], msg_b1[l], msg_g[l], msg_beta[l],
                      msg_W2[l], msg_b2[l], blk=4000)
        agg = _sc_scatter_add(m, dst, zeros)
        x = _gru(agg[0], agg[1], x, gru_Wi[l], gru_Wh[l], gru_bi[l],
                 gru_bh[l], ln_g[l], ln_beta[l], blk=2000)

    s, mx, cnt = _segment_reduce(x, batch, nseg, blk=1000)
    return _project(s, mx, cnt, out_W, out_b, out_g, out_beta)
